# baseline (speedup 1.0000x reference)
import jax
import jax.numpy as jnp
from jax import lax
from jax.experimental import pallas as pl
from jax.experimental.pallas import tpu as pltpu
from jax.experimental.pallas import tpu_sc as plsc

B, D, H, W = 8, 256, 32, 32
C = 2 * D
NC, NS, L = 2, 16, 16
NW = NC * NS
PW = C // NW
PLANE = H * W
SLAB = PW * PLANE


def _sc_body(row_hbm, col_hbm, out_hbm, slab_v, buf_v, sem):
    cid = lax.axis_index("c")
    sid = lax.axis_index("s")
    wid = sid * NC + cid
    half = wid // (NW // 2)
    c0 = (wid % (NW // 2)) * PW

    @pl.when(half == 0)
    def _():
        pltpu.sync_copy(col_hbm.at[pl.ds(0, H), pl.ds(c0, PW)], slab_v)

    @pl.when(half == 1)
    def _():
        pltpu.sync_copy(row_hbm.at[pl.ds(0, H), pl.ds(c0, PW)], slab_v)

    iota = lax.iota(jnp.int32, L)

    @pl.when(half == 0)
    def _():
        rows = [slab_v[r] for r in range(H)]
        accs = []
        for p in range(PW):
            acc0 = jnp.zeros((L,), jnp.float32)
            acc1 = jnp.zeros((L,), jnp.float32)
            for w in range(L):
                acc0 = jnp.where(iota == w, rows[w][p], acc0)
                acc1 = jnp.where(iota == w, rows[w + L][p], acc1)
            accs.append((acc0, acc1))

        def fill_col(h, carry):
            for p in range(PW):
                buf_v[p, h, pl.ds(0, L)] = accs[p][0]
                buf_v[p, h, pl.ds(L, L)] = accs[p][1]
            return carry

        lax.fori_loop(0, H, fill_col, 0)

    @pl.when(half == 1)
    def _():
        def fill_row(h, carry):
            rv = slab_v[h]
            for p in range(PW):
                sv = jnp.full((L,), rv[p], jnp.float32)
                buf_v[p, h, pl.ds(0, L)] = sv
                buf_v[p, h, pl.ds(L, L)] = sv
            return carry

        lax.fori_loop(0, H, fill_row, 0)

    copies = [
        pltpu.async_copy(buf_v, out_hbm.at[b, pl.ds(wid * PW, PW)], sem)
        for b in range(B)
    ]
    for cp in copies:
        cp.wait()


@jax.jit
def _pos_embed(row_embed, col_embed):
    mesh = plsc.VectorSubcoreMesh(core_axis_name="c", subcore_axis_name="s")
    out = pl.kernel(
        _sc_body,
        mesh=mesh,
        compiler_params=pltpu.CompilerParams(use_tc_tiling_on_sc=False),
        out_type=jax.ShapeDtypeStruct((B, C, H, W), jnp.float32),
        scratch_types=[
            pltpu.VMEM((H, PW), jnp.float32),
            pltpu.VMEM((PW, H, W), jnp.float32),
            pltpu.SemaphoreType.DMA,
        ],
    )(row_embed, col_embed)
    return out


def kernel(x, row_embed, col_embed):
    del x
    return _pos_embed(row_embed, col_embed)


# speedup vs baseline: 3.2758x; 3.2758x over previous
import jax
import jax.numpy as jnp
from jax import lax
from jax.experimental import pallas as pl
from jax.experimental.pallas import tpu as pltpu
from jax.experimental.pallas import tpu_sc as plsc

B, D, H, W = 8, 256, 32, 32
C = 2 * D
NC, NS, L = 2, 16, 16
NW = NC * NS
WT, CT = W // 8, C // 128


def _sc_body(row_hbm, col_hbm, out_hbm, col_v, row_v, plane_v, sem):
    cid = lax.axis_index("c")
    sid = lax.axis_index("s")
    h = sid * NC + cid

    pltpu.sync_copy(col_hbm.at[pl.ds(0, W)], col_v)
    pltpu.sync_copy(row_hbm.at[h], row_v)

    seg = [[row_v[pl.ds(ct * 128 + 16 * k, L)] for k in range(8)] for ct in range(2)]

    def build(wi, carry):
        for wt in range(WT):
            for ct in range(2):
                for k in range(8):
                    plane_v[wt, ct, wi, pl.ds(16 * k, L)] = col_v[
                        8 * wt + wi, pl.ds(128 * ct + 16 * k, L)
                    ]
            for ct in range(2):
                for k in range(8):
                    plane_v[wt, 2 + ct, wi, pl.ds(16 * k, L)] = seg[ct][k]
        return carry

    lax.fori_loop(0, 8, build, 0)

    copies = [
        pltpu.async_copy(plane_v, out_hbm.at[b, h], sem) for b in range(B)
    ]
    for cp in copies:
        cp.wait()


@jax.jit
def _pos_embed(row_embed, col_embed):
    mesh = plsc.VectorSubcoreMesh(core_axis_name="c", subcore_axis_name="s")
    out = pl.kernel(
        _sc_body,
        mesh=mesh,
        compiler_params=pltpu.CompilerParams(use_tc_tiling_on_sc=False),
        out_type=jax.ShapeDtypeStruct((B, H, WT, CT, 8, 128), jnp.float32),
        scratch_types=[
            pltpu.VMEM((W, D), jnp.float32),
            pltpu.VMEM((D,), jnp.float32),
            pltpu.VMEM((WT, CT, 8, 128), jnp.float32),
            pltpu.SemaphoreType.DMA,
        ],
    )(row_embed, col_embed)
    return out.transpose(0, 3, 5, 1, 2, 4).reshape(B, C, H, W)


def kernel(x, row_embed, col_embed):
    del x
    return _pos_embed(row_embed, col_embed)


# speedup vs baseline: 3.3516x; 1.0231x over previous
import jax
import jax.numpy as jnp
from jax import lax
from jax.experimental import pallas as pl
from jax.experimental.pallas import tpu as pltpu
from jax.experimental.pallas import tpu_sc as plsc

B, D, H, W = 8, 256, 32, 32
C = 2 * D
NC, NS, L = 2, 16, 16
NW = NC * NS
WT, CT = W // 8, C // 128


def _sc_body(row_hbm, col_hbm, out_hbm, col_v, row_v, plane_v, sem):
    cid = lax.axis_index("c")
    sid = lax.axis_index("s")
    h = sid * NC + cid

    pltpu.sync_copy(col_hbm.at[pl.ds(0, W)], col_v)
    pltpu.sync_copy(row_hbm.at[h], row_v)

    seg = [[row_v[pl.ds(ct * 128 + 16 * k, L)] for k in range(8)] for ct in range(2)]

    copies = []
    for wt in range(WT):
        def build(wi, carry, wt=wt):
            for ct in range(2):
                for k in range(8):
                    plane_v[wt, ct, wi, pl.ds(16 * k, L)] = col_v[
                        8 * wt + wi, pl.ds(128 * ct + 16 * k, L)
                    ]
            for ct in range(2):
                for k in range(8):
                    plane_v[wt, 2 + ct, wi, pl.ds(16 * k, L)] = seg[ct][k]
            return carry

        lax.fori_loop(0, 8, build, 0)
        copies.extend(
            pltpu.async_copy(plane_v.at[wt], out_hbm.at[b, h, wt], sem)
            for b in range(B)
        )
    for cp in copies:
        cp.wait()


@jax.jit
def _pos_embed(row_embed, col_embed):
    mesh = plsc.VectorSubcoreMesh(core_axis_name="c", subcore_axis_name="s")
    out = pl.kernel(
        _sc_body,
        mesh=mesh,
        compiler_params=pltpu.CompilerParams(use_tc_tiling_on_sc=False),
        out_type=jax.ShapeDtypeStruct((B, H, WT, CT, 8, 128), jnp.float32),
        scratch_types=[
            pltpu.VMEM((W, D), jnp.float32),
            pltpu.VMEM((D,), jnp.float32),
            pltpu.VMEM((WT, CT, 8, 128), jnp.float32),
            pltpu.SemaphoreType.DMA,
        ],
    )(row_embed, col_embed)
    return out.transpose(0, 3, 5, 1, 2, 4).reshape(B, C, H, W)


def kernel(x, row_embed, col_embed):
    del x
    return _pos_embed(row_embed, col_embed)
